# baseline (speedup 1.0000x reference)
import functools

import jax
import jax.numpy as jnp
from jax import lax
from jax.experimental import pallas as pl
from jax.experimental.pallas import tpu as pltpu
from jax.experimental.pallas import tpu_sc as plsc

_N = 4194304
_BINS = 512
_NC = 2
_NS = 16
_NW = _NC * _NS
_PER_W = _N // _NW
_CHUNK = 8192
_NBUF = 2
_NCHUNK = _PER_W // _CHUNK
_VECS = _CHUNK // 16
_LANES = 16
_NHIST = 4

_mesh = plsc.VectorSubcoreMesh(core_axis_name="c", subcore_axis_name="s")


@functools.partial(
    pl.kernel,
    out_type=jax.ShapeDtypeStruct((_NW, _BINS), jnp.float32),
    mesh=_mesh,
    compiler_params=pltpu.CompilerParams(needs_layout_passes=False,
                                         use_tc_tiling_on_sc=False),
    scratch_types=[
        pltpu.VMEM((_NBUF, _CHUNK), jnp.float32),
        pltpu.VMEM((_NBUF, _CHUNK), jnp.float32),
        pltpu.VMEM((_NBUF, _CHUNK), jnp.float32),
        pltpu.VMEM((_NBUF, _CHUNK), jnp.int32),
        pltpu.VMEM((_BINS * _LANES,), jnp.float32),
        pltpu.VMEM((_BINS * _LANES,), jnp.float32),
        pltpu.VMEM((_BINS * _LANES,), jnp.float32),
        pltpu.VMEM((_BINS * _LANES,), jnp.float32),
        pltpu.VMEM((_BINS,), jnp.float32),
        pltpu.SemaphoreType.DMA,
        pltpu.SemaphoreType.DMA,
    ],
)
def _sc_hist(in_hbm, tg_hbm, wt_hbm, dt_hbm, out_hbm,
             ibuf, tbuf, wbuf, dbuf, hist0, hist1, hist2, hist3, pibuf,
             sem0, sem1):
    hists = (hist0, hist1, hist2, hist3)
    wid = lax.axis_index("s") * _NC + lax.axis_index("c")
    base = wid * _PER_W
    sems = (sem0, sem1)

    zero16 = jnp.zeros((16,), jnp.float32)

    _hwords = _LANES * _BINS // 16

    @plsc.parallel_loop(0, _hwords)
    def _zero(j):
        c = j * 16
        for h in hists:
            h[pl.ds(c, 16)] = zero16

    lane = lax.broadcasted_iota(jnp.int32, (16,), 0)

    def issue(k, slot):
        off = base + k * _CHUNK
        sem = sems[slot]
        return [
            pltpu.async_copy(in_hbm.at[pl.ds(off, _CHUNK)], ibuf.at[slot], sem),
            pltpu.async_copy(tg_hbm.at[pl.ds(off, _CHUNK)], tbuf.at[slot], sem),
            pltpu.async_copy(wt_hbm.at[pl.ds(off, _CHUNK)], wbuf.at[slot], sem),
            pltpu.async_copy(dt_hbm.at[pl.ds(off, _CHUNK)], dbuf.at[slot], sem),
        ]

    def compute(slot):
        @plsc.parallel_loop(0, _VECS // _NHIST, unroll=2)
        def _body(j):
            base_off = j * (16 * _NHIST)
            for u in range(_NHIST):
                off = base_off + u * 16
                iv = ibuf[slot, pl.ds(off, 16)]
                tv = tbuf[slot, pl.ds(off, 16)]
                wv = wbuf[slot, pl.ds(off, 16)]
                dv = dbuf[slot, pl.ds(off, 16)]
                plsc.addupdate_scatter(hists[u], [dv * _LANES + lane],
                                       iv * tv * wv)

    inflight = [issue(k, k) for k in range(_NBUF - 1)]
    for k in range(_NCHUNK):
        slot = k % _NBUF
        for c in inflight.pop(0):
            c.wait()
        if k + _NBUF - 1 < _NCHUNK:
            inflight.append(issue(k + _NBUF - 1, (k + _NBUF - 1) % _NBUF))
        compute(slot)

    @plsc.parallel_loop(0, _BINS // 16)
    def _reduce(g):
        acc = zero16
        for k in range(16):
            off = (g * 16 + k) * _LANES
            v = (hist0[pl.ds(off, 16)] + hist1[pl.ds(off, 16)]
                 + hist2[pl.ds(off, 16)] + hist3[pl.ds(off, 16)])
            acc = jnp.where(lane == k, jnp.sum(v), acc)
        pibuf[pl.ds(g * 16, 16)] = acc

    pltpu.sync_copy(pibuf, out_hbm.at[wid])


def _tc_reduce_body(p_ref, o_ref):
    x = p_ref[...]
    pi = jnp.sum(x, axis=0, keepdims=True)
    s = jnp.sum(pi)
    ss = jnp.sum(pi * pi)
    o_ref[0, 0] = -(s * s) / ss


_tc_reduce = pl.pallas_call(
    _tc_reduce_body,
    out_shape=jax.ShapeDtypeStruct((1, 1), jnp.float32),
    out_specs=pl.BlockSpec(memory_space=pltpu.SMEM),
)


def kernel(inputs, targets, weight, date, scaling):
    partials = _sc_hist(inputs.reshape(-1), targets.reshape(-1),
                        weight.reshape(-1), date)
    pi = jnp.sum(partials, axis=0)
    s = jnp.sum(pi)
    ss = jnp.sum(pi * pi)
    return -(s * s) / ss


# speedup vs baseline: 1.2092x; 1.2092x over previous
import functools

import jax
import jax.numpy as jnp
from jax import lax
from jax.experimental import pallas as pl
from jax.experimental.pallas import tpu as pltpu
from jax.experimental.pallas import tpu_sc as plsc

_N = 4194304
_BINS = 512
_NC = 2
_NS = 16
_NW = _NC * _NS
_PER_W = _N // _NW
_CHUNK = 8192
_NBUF = 3
_NCHUNK = _PER_W // _CHUNK
_VECS = _CHUNK // 16
_LANES = 16
_NHIST = 2

_mesh = plsc.VectorSubcoreMesh(core_axis_name="c", subcore_axis_name="s")


@functools.partial(
    pl.kernel,
    out_type=jax.ShapeDtypeStruct((_NW, _BINS), jnp.float32),
    mesh=_mesh,
    compiler_params=pltpu.CompilerParams(needs_layout_passes=False,
                                         use_tc_tiling_on_sc=False),
    scratch_types=[
        pltpu.VMEM((_NBUF, _CHUNK), jnp.float32),
        pltpu.VMEM((_NBUF, _CHUNK), jnp.float32),
        pltpu.VMEM((_NBUF, _CHUNK), jnp.float32),
        pltpu.VMEM((_NBUF, _CHUNK), jnp.int32),
        pltpu.VMEM((_BINS * _LANES,), jnp.float32),
        pltpu.VMEM((_BINS * _LANES,), jnp.float32),
        pltpu.VMEM((_BINS,), jnp.float32),
        pltpu.SemaphoreType.DMA,
        pltpu.SemaphoreType.DMA,
        pltpu.SemaphoreType.DMA,
    ],
)
def _sc_hist(in_hbm, tg_hbm, wt_hbm, dt_hbm, out_hbm,
             ibuf, tbuf, wbuf, dbuf, hist0, hist1, pibuf,
             sem0, sem1, sem2):
    hists = (hist0, hist1)
    wid = lax.axis_index("s") * _NC + lax.axis_index("c")
    base = wid * _PER_W
    sems = (sem0, sem1, sem2)

    zero16 = jnp.zeros((16,), jnp.float32)

    _hwords = _LANES * _BINS // 16

    @plsc.parallel_loop(0, _hwords)
    def _zero(j):
        c = j * 16
        for h in hists:
            h[pl.ds(c, 16)] = zero16

    lane = lax.broadcasted_iota(jnp.int32, (16,), 0)

    def issue(k, slot):
        off = base + k * _CHUNK
        sem = sems[slot]
        return [
            pltpu.async_copy(in_hbm.at[pl.ds(off, _CHUNK)], ibuf.at[slot], sem),
            pltpu.async_copy(tg_hbm.at[pl.ds(off, _CHUNK)], tbuf.at[slot], sem),
            pltpu.async_copy(wt_hbm.at[pl.ds(off, _CHUNK)], wbuf.at[slot], sem),
            pltpu.async_copy(dt_hbm.at[pl.ds(off, _CHUNK)], dbuf.at[slot], sem),
        ]

    def compute(slot):
        @plsc.parallel_loop(0, _VECS // _NHIST, unroll=2)
        def _body(j):
            base_off = j * (16 * _NHIST)
            for u in range(_NHIST):
                off = base_off + u * 16
                iv = ibuf[slot, pl.ds(off, 16)]
                tv = tbuf[slot, pl.ds(off, 16)]
                wv = wbuf[slot, pl.ds(off, 16)]
                dv = dbuf[slot, pl.ds(off, 16)]
                plsc.addupdate_scatter(hists[u], [dv * _LANES + lane],
                                       iv * tv * wv)

    inflight = [issue(k, k) for k in range(_NBUF - 1)]
    for k in range(_NCHUNK):
        slot = k % _NBUF
        for c in inflight.pop(0):
            c.wait()
        if k + _NBUF - 1 < _NCHUNK:
            inflight.append(issue(k + _NBUF - 1, (k + _NBUF - 1) % _NBUF))
        compute(slot)

    @plsc.parallel_loop(0, _BINS // 16)
    def _reduce(g):
        acc = zero16
        for k in range(16):
            off = (g * 16 + k) * _LANES
            v = hist0[pl.ds(off, 16)] + hist1[pl.ds(off, 16)]
            acc = jnp.where(lane == k, jnp.sum(v), acc)
        pibuf[pl.ds(g * 16, 16)] = acc

    pltpu.sync_copy(pibuf, out_hbm.at[wid])


def _tc_reduce_body(p_ref, o_ref):
    x = p_ref[...]
    pi = jnp.sum(x, axis=0, keepdims=True)
    s = jnp.sum(pi)
    ss = jnp.sum(pi * pi)
    o_ref[0, 0] = -(s * s) / ss


_tc_reduce = pl.pallas_call(
    _tc_reduce_body,
    out_shape=jax.ShapeDtypeStruct((1, 1), jnp.float32),
    out_specs=pl.BlockSpec(memory_space=pltpu.SMEM),
)


def kernel(inputs, targets, weight, date, scaling):
    partials = _sc_hist(inputs.reshape(-1), targets.reshape(-1),
                        weight.reshape(-1), date)
    return _tc_reduce(partials)[0, 0]
